# Initial kernel scaffold; baseline (speedup 1.0000x reference)
import jax
import jax.numpy as jnp
from jax.experimental import pallas as pl


def kernel(x, params, edge_index, batch):
    raise NotImplementedError("write your pallas kernel here")


# speedup vs baseline: 3.5884x; 3.5884x over previous
import functools

import jax
import jax.numpy as jnp
from jax import lax
from jax.experimental import pallas as pl
from jax.experimental.pallas import tpu as pltpu
from jax.experimental.pallas import tpu_sc as plsc

_N = 10000
_E = 320000
_D = 128
_H = 128
_MH = 64
_G = 64

_NC = 2
_NS = 16
_NT = _NC * _NS
_EW = _E // _NT
_CH = 64
_NCH = _EW // _CH
_TAIL = _EW - _NCH * _CH
_RPT = 624


def _dot(a, b):
    return lax.dot_general(a, b, (((1,), (0,)), ((), ())),
                           precision=lax.Precision.HIGHEST,
                           preferred_element_type=jnp.float32)


def _make_sc_msg(w):
    out_type = [jax.ShapeDtypeStruct((_NC, _N, w), jnp.float32)]
    scratch = [
        pltpu.VMEM((_CH,), jnp.int32),
        pltpu.VMEM((_CH,), jnp.int32),
        pltpu.VMEM((_CH, w), jnp.float32),
        pltpu.VMEM((_TAIL,), jnp.int32),
        pltpu.VMEM((_TAIL,), jnp.int32),
        pltpu.VMEM((_TAIL, w), jnp.float32),
        pltpu.VMEM((48, w), jnp.float32),
        pltpu.SemaphoreType.DMA,
        pltpu.VMEM_SHARED((_N, w), jnp.float32),
    ]

    mesh = plsc.VectorSubcoreMesh(core_axis_name="c", subcore_axis_name="s")

    @functools.partial(pl.kernel, mesh=mesh, out_type=out_type,
                       scratch_types=scratch)
    def sc_msg(h_hbm, src_hbm, dst_hbm, msg_out, src_v, dst_v, rows_v,
               src_t, dst_t, rows_t, stage_v, gsem, msg_sh):
        cid = lax.axis_index("c")
        sid = lax.axis_index("s")
        wid = cid * _NS + sid
        row0 = sid * _RPT

        zv = jnp.zeros((16,), jnp.float32)

        @pl.loop(0, _TAIL)
        def _(i):
            @pl.loop(0, w // 16)
            def _(j):
                rows_t[i, pl.ds(j * 16, 16)] = zv

        @pl.loop(0, _RPT // 16)
        def _(i):
            pltpu.sync_copy(rows_t, msg_sh.at[pl.ds(row0 + i * 16, 16)])

        @pl.when(sid == _NS - 1)
        def _():
            pltpu.sync_copy(rows_t, msg_sh.at[pl.ds(_NS * _RPT, 16)])

        plsc.subcore_barrier()

        base = wid * _EW

        @pl.loop(0, _NCH)
        def _(c):
            off = base + c * _CH
            pltpu.sync_copy(src_hbm.at[pl.ds(off, _CH)], src_v)
            pltpu.sync_copy(dst_hbm.at[pl.ds(off, _CH)], dst_v)
            pltpu.async_copy(h_hbm.at[src_v], rows_v, gsem).wait()
            pltpu.sync_copy(rows_v, msg_sh.at[dst_v], add=True)

        toff = base + _NCH * _CH
        pltpu.sync_copy(src_hbm.at[pl.ds(toff, _TAIL)], src_t)
        pltpu.sync_copy(dst_hbm.at[pl.ds(toff, _TAIL)], dst_t)
        pltpu.async_copy(h_hbm.at[src_t], rows_t, gsem).wait()
        pltpu.sync_copy(rows_t, msg_sh.at[dst_t], add=True)

        plsc.subcore_barrier()

        sr = 48

        @pl.loop(0, _RPT // 48)
        def _(i):
            r = row0 + i * sr
            pltpu.sync_copy(msg_sh.at[pl.ds(r, sr)], stage_v)
            pltpu.sync_copy(stage_v, msg_out.at[cid, pl.ds(r, sr)])

        @pl.when(sid == _NS - 1)
        def _():
            pltpu.sync_copy(msg_sh.at[pl.ds(_NS * _RPT, 16)], rows_t)
            pltpu.sync_copy(rows_t, msg_out.at[cid, pl.ds(_NS * _RPT, 16)])

    return sc_msg


_sc_msg = _make_sc_msg(_H)


def _make_sc_deg():
    mesh = plsc.VectorSubcoreMesh(core_axis_name="c", subcore_axis_name="s")

    @functools.partial(
        pl.kernel, mesh=mesh,
        out_type=[jax.ShapeDtypeStruct((_NC, _N, _H), jnp.float32)],
        scratch_types=[
            pltpu.VMEM((_CH,), jnp.int32),
            pltpu.VMEM((_CH, _H), jnp.float32),
            pltpu.VMEM((_TAIL,), jnp.int32),
            pltpu.VMEM((_TAIL, _H), jnp.float32),
            pltpu.VMEM((48, _H), jnp.float32),
            pltpu.VMEM_SHARED((_N, _H), jnp.float32),
        ])
    def sc_deg(dst_hbm, deg_out, dst_v, ones_v, dst_t, z16, stage_v, deg_sh):
        cid = lax.axis_index("c")
        sid = lax.axis_index("s")
        wid = cid * _NS + sid
        row0 = sid * _RPT

        zv = jnp.zeros((16,), jnp.float32)
        ov = jnp.full((16,), 1.0, jnp.float32)

        @pl.loop(0, _TAIL)
        def _(i):
            @pl.loop(0, _H // 16)
            def _(j):
                z16[i, pl.ds(j * 16, 16)] = zv

        @pl.loop(0, _CH)
        def _(i):
            @pl.loop(0, _H // 16)
            def _(j):
                ones_v[i, pl.ds(j * 16, 16)] = ov

        @pl.loop(0, _RPT // 16)
        def _(i):
            pltpu.sync_copy(z16, deg_sh.at[pl.ds(row0 + i * 16, 16)])

        @pl.when(sid == _NS - 1)
        def _():
            pltpu.sync_copy(z16, deg_sh.at[pl.ds(_NS * _RPT, 16)])

        plsc.subcore_barrier()

        base = wid * _EW

        @pl.loop(0, _NCH)
        def _(c):
            pltpu.sync_copy(dst_hbm.at[pl.ds(base + c * _CH, _CH)], dst_v)
            pltpu.sync_copy(ones_v, deg_sh.at[dst_v], add=True)

        pltpu.sync_copy(dst_hbm.at[pl.ds(base + _NCH * _CH, _TAIL)], dst_t)
        pltpu.sync_copy(ones_v.at[pl.ds(0, _TAIL)], deg_sh.at[dst_t], add=True)

        plsc.subcore_barrier()

        sr = 48

        @pl.loop(0, _RPT // 48)
        def _(i):
            r = row0 + i * sr
            pltpu.sync_copy(deg_sh.at[pl.ds(r, sr)], stage_v)
            pltpu.sync_copy(stage_v, deg_out.at[cid, pl.ds(r, sr)])

        @pl.when(sid == _NS - 1)
        def _():
            pltpu.sync_copy(deg_sh.at[pl.ds(_NS * _RPT, 16)], z16)
            pltpu.sync_copy(z16, deg_out.at[cid, pl.ds(_NS * _RPT, 16)])

    return sc_deg


_sc_deg = _make_sc_deg()


_RB = 1000
_NB = _N // _RB


def _ln(x, g, b, eps=1e-5):
    m = jnp.mean(x, axis=-1, keepdims=True)
    v = jnp.mean((x - m) ** 2, axis=-1, keepdims=True)
    return (x - m) / jnp.sqrt(v + eps) * g + b


def _encode_body(x_ref, w0, b0, w1, b1, w2, b2, g, b, w3, b3, out_ref):
    x = x_ref[...]
    h = jnp.maximum(_dot(x, w0[...]) + b0[...], 0.0)
    h = jnp.maximum(_dot(h, w1[...]) + b1[...], 0.0)
    h = jnp.maximum(_dot(h, w2[...]) + b2[...], 0.0)
    h = _ln(h, g[...], b[...])
    out_ref[...] = _dot(h, w3[...]) + b3[...]


def _encode(x, p):
    full = lambda s: pl.BlockSpec(s, lambda i: (0,) * len(s))
    return pl.pallas_call(
        _encode_body,
        grid=(_NB,),
        in_specs=[
            pl.BlockSpec((_RB, _D), lambda i: (i, 0)),
            full((_D, _MH)), full((1, _MH)),
            full((_MH, _MH)), full((1, _MH)),
            full((_MH, _MH)), full((1, _MH)),
            full((1, _MH)), full((1, _MH)),
            full((_MH, _H)), full((1, _H)),
        ],
        out_specs=pl.BlockSpec((_RB, _H), lambda i: (i, 0)),
        out_shape=jax.ShapeDtypeStruct((_N, _H), jnp.float32),
    )(x, p['w0'], p['b0'].reshape(1, -1), p['w1'], p['b1'].reshape(1, -1),
      p['w2'], p['b2'].reshape(1, -1), p['ln_g'].reshape(1, -1),
      p['ln_b'].reshape(1, -1), p['w3'], p['b3'].reshape(1, -1))


def _combine1_body(msgp_ref, degp_ref, h_ref, wl_ref, wr_ref, bl_ref,
                   out_ref, inv_ref):
    m = msgp_ref[...]
    d = degp_ref[...]
    msg = m[0] + m[1]
    deg = (d[0] + d[1])[:, 0:1]
    inv = 1.0 / jnp.maximum(deg, 1.0)
    mean = msg * inv
    inv_ref[...] = inv
    out_ref[...] = jnp.maximum(
        _dot(mean, wl_ref[...]) + _dot(h_ref[...], wr_ref[...]) + bl_ref[...],
        0.0)


def _combine1(msgp, degp, h, p):
    full = lambda s: pl.BlockSpec(s, lambda i: (0,) * len(s))
    return pl.pallas_call(
        _combine1_body,
        grid=(_NB,),
        in_specs=[
            pl.BlockSpec((_NC, _RB, _H), lambda i: (0, i, 0)),
            pl.BlockSpec((_NC, _RB, _H), lambda i: (0, i, 0)),
            pl.BlockSpec((_RB, _H), lambda i: (i, 0)),
            full((_H, _H)), full((_H, _H)), full((1, _H)),
        ],
        out_specs=[pl.BlockSpec((_RB, _H), lambda i: (i, 0)),
                   pl.BlockSpec((_RB, 1), lambda i: (i, 0))],
        out_shape=[jax.ShapeDtypeStruct((_N, _H), jnp.float32),
                   jax.ShapeDtypeStruct((_N, 1), jnp.float32)],
    )(msgp, degp, h, p['wl'], p['wr'], p['bl'].reshape(1, -1))


def _combine_body(msgp_ref, inv_ref, h_ref, wl_ref, wr_ref, bl_ref, out_ref):
    m = msgp_ref[...]
    mean = (m[0] + m[1]) * inv_ref[...]
    out_ref[...] = jnp.maximum(
        _dot(mean, wl_ref[...]) + _dot(h_ref[...], wr_ref[...]) + bl_ref[...],
        0.0)


def _combine(msgp, inv, h, p):
    full = lambda s: pl.BlockSpec(s, lambda i: (0,) * len(s))
    return pl.pallas_call(
        _combine_body,
        grid=(_NB,),
        in_specs=[
            pl.BlockSpec((_NC, _RB, _H), lambda i: (0, i, 0)),
            pl.BlockSpec((_RB, 1), lambda i: (i, 0)),
            pl.BlockSpec((_RB, _H), lambda i: (i, 0)),
            full((_H, _H)), full((_H, _H)), full((1, _H)),
        ],
        out_specs=pl.BlockSpec((_RB, _H), lambda i: (i, 0)),
        out_shape=jax.ShapeDtypeStruct((_N, _H), jnp.float32),
    )(msgp, inv, h, p['wl'], p['wr'], p['bl'].reshape(1, -1))


def _pool_body(h_ref, batch_ref, out_ref):
    i = pl.program_id(0)

    @pl.when(i == 0)
    def _():
        out_ref[...] = jnp.zeros_like(out_ref)

    b = batch_ref[0, 0, :]
    mask = (b[:, None] == lax.broadcasted_iota(jnp.int32, (1, _G), 1)
            ).astype(jnp.float32)
    out_ref[...] += lax.dot_general(
        mask, h_ref[...], (((0,), (0,)), ((), ())),
        precision=lax.Precision.HIGHEST,
        preferred_element_type=jnp.float32)


def _pool(h, batch3):
    return pl.pallas_call(
        _pool_body,
        grid=(_NB,),
        in_specs=[
            pl.BlockSpec((_RB, _H), lambda i: (i, 0)),
            pl.BlockSpec((1, 1, _RB), lambda i: (i, 0, 0)),
        ],
        out_specs=pl.BlockSpec((_G, _H), lambda i: (0, 0)),
        out_shape=jax.ShapeDtypeStruct((_G, _H), jnp.float32),
    )(h, batch3)


def _decode_body(p_ref, g0a, c0a, w0a, b0a, g1a, c1a, w1a, b1a,
                 g0b, c0b, w0b, b0b, g1b, c1b, w1b, b1b, out_ref):
    x = p_ref[...]
    outs = []
    for (g0, c0, w0, b0, g1, c1, w1, b1) in (
            (g0a, c0a, w0a, b0a, g1a, c1a, w1a, b1a),
            (g0b, c0b, w0b, b0b, g1b, c1b, w1b, b1b)):
        x1 = jnp.maximum(_dot(_ln(x, g0[...], c0[...]), w0[...]) + b0[...],
                         0.0)
        x2 = jnp.maximum(_dot(_ln(x1, g1[...], c1[...]), w1[...]) + b1[...],
                         0.0)
        outs.append(x2)
    out_ref[...] = jnp.concatenate(outs, axis=1)


def _decode(pooled, dec):
    args = [pooled]
    for head in dec:
        for lp in head:
            args += [lp['ln_g'].reshape(1, -1), lp['ln_b'].reshape(1, -1),
                     lp['w'], lp['b'].reshape(1, -1)]
    return pl.pallas_call(
        _decode_body,
        out_shape=jax.ShapeDtypeStruct((_G, 2), jnp.float32),
    )(*args)


def kernel(x, params, edge_index, batch):
    src = edge_index[0]
    dst = edge_index[1]
    r = _sc_deg(dst)
    degp = r[0] if isinstance(r, (list, tuple)) else r
    h = _encode(x, params['enc'])
    r = _sc_msg(h, src, dst)
    msgp = r[0] if isinstance(r, (list, tuple)) else r
    h, inv = _combine1(msgp, degp, h, params['convs'][0])
    for cp in params['convs'][1:]:
        r = _sc_msg(h, src, dst)
        msgp = r[0] if isinstance(r, (list, tuple)) else r
        h = _combine(msgp, inv, h, cp)
    pooled = _pool(h, batch.reshape(_NB, 1, _RB))
    return _decode(pooled, params['dec'])


# speedup vs baseline: 5.8347x; 1.6260x over previous
import functools

import jax
import jax.numpy as jnp
from jax import lax
from jax.experimental import pallas as pl
from jax.experimental.pallas import tpu as pltpu
from jax.experimental.pallas import tpu_sc as plsc

_N = 10000
_E = 320000
_D = 128
_H = 128
_MH = 64
_G = 64

_NC = 2
_NS = 16
_NT = _NC * _NS
_EW = _E // _NT
_CH = 64
_NCH = _EW // _CH
_TAIL = _EW - _NCH * _CH
_RPT = 624


def _dot(a, b):
    return lax.dot_general(a, b, (((1,), (0,)), ((), ())),
                           preferred_element_type=jnp.float32)


def _make_sc_msg(w):
    out_type = [jax.ShapeDtypeStruct((_NC, _N, w), jnp.float32)]
    scratch = [
        pltpu.VMEM((_CH,), jnp.int32),
        pltpu.VMEM((_CH,), jnp.int32),
        pltpu.VMEM((_CH, w), jnp.float32),
        pltpu.VMEM((_CH,), jnp.int32),
        pltpu.VMEM((_CH,), jnp.int32),
        pltpu.VMEM((_CH, w), jnp.float32),
        pltpu.VMEM((_TAIL,), jnp.int32),
        pltpu.VMEM((_TAIL,), jnp.int32),
        pltpu.VMEM((_TAIL, w), jnp.float32),
        pltpu.VMEM((48, w), jnp.float32),
        pltpu.SemaphoreType.DMA,
        pltpu.SemaphoreType.DMA,
        pltpu.VMEM_SHARED((_N, w), jnp.float32),
    ]

    mesh = plsc.VectorSubcoreMesh(core_axis_name="c", subcore_axis_name="s")

    @functools.partial(pl.kernel, mesh=mesh, out_type=out_type,
                       scratch_types=scratch)
    def sc_msg(h_hbm, src_hbm, dst_hbm, msg_out, src_v0, dst_v0, rows_v0,
               src_v1, dst_v1, rows_v1, src_t, dst_t, rows_t, stage_v,
               gsem0, gsem1, msg_sh):
        cid = lax.axis_index("c")
        sid = lax.axis_index("s")
        wid = cid * _NS + sid
        row0 = sid * _RPT

        zv = jnp.zeros((16,), jnp.float32)

        @pl.loop(0, _TAIL)
        def _(i):
            @pl.loop(0, w // 16)
            def _(j):
                rows_t[i, pl.ds(j * 16, 16)] = zv

        @pl.loop(0, _RPT // 16)
        def _(i):
            pltpu.sync_copy(rows_t, msg_sh.at[pl.ds(row0 + i * 16, 16)])

        @pl.when(sid == _NS - 1)
        def _():
            pltpu.sync_copy(rows_t, msg_sh.at[pl.ds(_NS * _RPT, 16)])

        plsc.subcore_barrier()

        base = wid * _EW
        npair = _NCH // 2

        def load_idx(c, sv, dv):
            off = base + c * _CH
            pltpu.sync_copy(src_hbm.at[pl.ds(off, _CH)], sv)
            pltpu.sync_copy(dst_hbm.at[pl.ds(off, _CH)], dv)

        load_idx(0, src_v0, dst_v0)
        pltpu.async_copy(h_hbm.at[src_v0], rows_v0, gsem0)

        @pl.loop(0, npair)
        def _(i2):
            c0 = 2 * i2
            load_idx(c0 + 1, src_v1, dst_v1)
            pltpu.async_copy(h_hbm.at[src_v1], rows_v1, gsem1)
            pltpu.make_async_copy(h_hbm.at[src_v0], rows_v0, gsem0).wait()
            pltpu.sync_copy(rows_v0, msg_sh.at[dst_v0], add=True)

            @pl.when(i2 < npair - 1)
            def _():
                load_idx(c0 + 2, src_v0, dst_v0)
                pltpu.async_copy(h_hbm.at[src_v0], rows_v0, gsem0)

            pltpu.make_async_copy(h_hbm.at[src_v1], rows_v1, gsem1).wait()
            pltpu.sync_copy(rows_v1, msg_sh.at[dst_v1], add=True)

        toff = base + _NCH * _CH
        pltpu.sync_copy(src_hbm.at[pl.ds(toff, _TAIL)], src_t)
        pltpu.sync_copy(dst_hbm.at[pl.ds(toff, _TAIL)], dst_t)
        pltpu.async_copy(h_hbm.at[src_t], rows_t, gsem0).wait()
        pltpu.sync_copy(rows_t, msg_sh.at[dst_t], add=True)

        plsc.subcore_barrier()

        sr = 48

        @pl.loop(0, _RPT // 48)
        def _(i):
            r = row0 + i * sr
            pltpu.sync_copy(msg_sh.at[pl.ds(r, sr)], stage_v)
            pltpu.sync_copy(stage_v, msg_out.at[cid, pl.ds(r, sr)])

        @pl.when(sid == _NS - 1)
        def _():
            pltpu.sync_copy(msg_sh.at[pl.ds(_NS * _RPT, 16)], rows_t)
            pltpu.sync_copy(rows_t, msg_out.at[cid, pl.ds(_NS * _RPT, 16)])

    return sc_msg


_sc_msg = _make_sc_msg(_H)


def _make_sc_deg():
    mesh = plsc.VectorSubcoreMesh(core_axis_name="c", subcore_axis_name="s")

    @functools.partial(
        pl.kernel, mesh=mesh,
        out_type=[jax.ShapeDtypeStruct((_NC, _N, _H), jnp.float32)],
        scratch_types=[
            pltpu.VMEM((_CH,), jnp.int32),
            pltpu.VMEM((_CH,), jnp.int32),
            pltpu.VMEM((_CH, _H), jnp.float32),
            pltpu.VMEM((_TAIL,), jnp.int32),
            pltpu.VMEM((_TAIL, _H), jnp.float32),
            pltpu.VMEM((48, _H), jnp.float32),
            pltpu.SemaphoreType.DMA,
            pltpu.SemaphoreType.DMA,
            pltpu.VMEM_SHARED((_N, _H), jnp.float32),
        ])
    def sc_deg(dst_hbm, deg_out, dst_v0, dst_v1, ones_v, dst_t, z16, stage_v,
               ssem0, ssem1, deg_sh):
        cid = lax.axis_index("c")
        sid = lax.axis_index("s")
        wid = cid * _NS + sid
        row0 = sid * _RPT

        zv = jnp.zeros((16,), jnp.float32)
        ov = jnp.full((16,), 1.0, jnp.float32)

        @pl.loop(0, _TAIL)
        def _(i):
            @pl.loop(0, _H // 16)
            def _(j):
                z16[i, pl.ds(j * 16, 16)] = zv

        @pl.loop(0, _CH)
        def _(i):
            @pl.loop(0, _H // 16)
            def _(j):
                ones_v[i, pl.ds(j * 16, 16)] = ov

        @pl.loop(0, _RPT // 16)
        def _(i):
            pltpu.sync_copy(z16, deg_sh.at[pl.ds(row0 + i * 16, 16)])

        @pl.when(sid == _NS - 1)
        def _():
            pltpu.sync_copy(z16, deg_sh.at[pl.ds(_NS * _RPT, 16)])

        plsc.subcore_barrier()

        base = wid * _EW
        npair = _NCH // 2

        pltpu.sync_copy(dst_hbm.at[pl.ds(base, _CH)], dst_v0)
        pltpu.async_copy(ones_v, deg_sh.at[dst_v0], ssem0, add=True)

        @pl.loop(0, npair)
        def _(i2):
            c0 = 2 * i2
            pltpu.sync_copy(dst_hbm.at[pl.ds(base + (c0 + 1) * _CH, _CH)],
                            dst_v1)
            pltpu.async_copy(ones_v, deg_sh.at[dst_v1], ssem1, add=True)
            pltpu.make_async_copy(ones_v, deg_sh.at[dst_v0], ssem0).wait()

            @pl.when(i2 < npair - 1)
            def _():
                pltpu.sync_copy(dst_hbm.at[pl.ds(base + (c0 + 2) * _CH, _CH)],
                                dst_v0)
                pltpu.async_copy(ones_v, deg_sh.at[dst_v0], ssem0, add=True)

            pltpu.make_async_copy(ones_v, deg_sh.at[dst_v1], ssem1).wait()

        pltpu.sync_copy(dst_hbm.at[pl.ds(base + _NCH * _CH, _TAIL)], dst_t)
        pltpu.sync_copy(ones_v.at[pl.ds(0, _TAIL)], deg_sh.at[dst_t], add=True)

        plsc.subcore_barrier()

        sr = 48

        @pl.loop(0, _RPT // 48)
        def _(i):
            r = row0 + i * sr
            pltpu.sync_copy(deg_sh.at[pl.ds(r, sr)], stage_v)
            pltpu.sync_copy(stage_v, deg_out.at[cid, pl.ds(r, sr)])

        @pl.when(sid == _NS - 1)
        def _():
            pltpu.sync_copy(deg_sh.at[pl.ds(_NS * _RPT, 16)], z16)
            pltpu.sync_copy(z16, deg_out.at[cid, pl.ds(_NS * _RPT, 16)])

    return sc_deg


_sc_deg = _make_sc_deg()


_RB = 1000
_NB = _N // _RB


def _ln(x, g, b, eps=1e-5):
    m = jnp.mean(x, axis=-1, keepdims=True)
    v = jnp.mean((x - m) ** 2, axis=-1, keepdims=True)
    return (x - m) / jnp.sqrt(v + eps) * g + b


def _enc_pre_body(x_ref, w0, b0, w1, b1, w2, b2, out_ref):
    x = x_ref[...]
    h = jnp.maximum(_dot(x, w0[...]) + b0[...], 0.0)
    h = jnp.maximum(_dot(h, w1[...]) + b1[...], 0.0)
    out_ref[...] = jnp.maximum(_dot(h, w2[...]) + b2[...], 0.0)


def _enc_post_body(x_ref, w3, b3, out_ref):
    out_ref[...] = _dot(x_ref[...], w3[...]) + b3[...]


def _encode(x, p):
    full = lambda s: pl.BlockSpec(s, lambda i: (0,) * len(s))
    pre = pl.pallas_call(
        _enc_pre_body,
        grid=(_NB,),
        in_specs=[
            pl.BlockSpec((_RB, _D), lambda i: (i, 0)),
            full((_D, _MH)), full((1, _MH)),
            full((_MH, _MH)), full((1, _MH)),
            full((_MH, _MH)), full((1, _MH)),
        ],
        out_specs=pl.BlockSpec((_RB, _MH), lambda i: (i, 0)),
        out_shape=jax.ShapeDtypeStruct((_N, _MH), jnp.float32),
    )(x, p['w0'], p['b0'].reshape(1, -1), p['w1'], p['b1'].reshape(1, -1),
      p['w2'], p['b2'].reshape(1, -1))
    ln = _ln(pre, p['ln_g'], p['ln_b'])
    return pl.pallas_call(
        _enc_post_body,
        grid=(_NB,),
        in_specs=[
            pl.BlockSpec((_RB, _MH), lambda i: (i, 0)),
            full((_MH, _H)), full((1, _H)),
        ],
        out_specs=pl.BlockSpec((_RB, _H), lambda i: (i, 0)),
        out_shape=jax.ShapeDtypeStruct((_N, _H), jnp.float32),
    )(ln, p['w3'], p['b3'].reshape(1, -1))


def _combine1_body(msgp_ref, degp_ref, h_ref, wl_ref, wr_ref, bl_ref,
                   out_ref, deg_ref):
    m = msgp_ref[...]
    d = degp_ref[...]
    msg = m[0] + m[1]
    deg = (d[0] + d[1])[:, 0:1]
    deg_ref[...] = deg
    mean = msg / jnp.maximum(deg, 1.0)
    out_ref[...] = jnp.maximum(
        _dot(mean, wl_ref[...]) + bl_ref[...] + _dot(h_ref[...], wr_ref[...]),
        0.0)


def _combine1(msgp, degp, h, p):
    full = lambda s: pl.BlockSpec(s, lambda i: (0,) * len(s))
    return pl.pallas_call(
        _combine1_body,
        grid=(_NB,),
        in_specs=[
            pl.BlockSpec((_NC, _RB, _H), lambda i: (0, i, 0)),
            pl.BlockSpec((_NC, _RB, _H), lambda i: (0, i, 0)),
            pl.BlockSpec((_RB, _H), lambda i: (i, 0)),
            full((_H, _H)), full((_H, _H)), full((1, _H)),
        ],
        out_specs=[pl.BlockSpec((_RB, _H), lambda i: (i, 0)),
                   pl.BlockSpec((_RB, 1), lambda i: (i, 0))],
        out_shape=[jax.ShapeDtypeStruct((_N, _H), jnp.float32),
                   jax.ShapeDtypeStruct((_N, 1), jnp.float32)],
    )(msgp, degp, h, p['wl'], p['wr'], p['bl'].reshape(1, -1))


def _combine_body(msgp_ref, deg_ref, h_ref, wl_ref, wr_ref, bl_ref, out_ref):
    m = msgp_ref[...]
    mean = (m[0] + m[1]) / jnp.maximum(deg_ref[...], 1.0)
    out_ref[...] = jnp.maximum(
        _dot(mean, wl_ref[...]) + bl_ref[...] + _dot(h_ref[...], wr_ref[...]),
        0.0)


def _combine(msgp, deg, h, p):
    full = lambda s: pl.BlockSpec(s, lambda i: (0,) * len(s))
    return pl.pallas_call(
        _combine_body,
        grid=(_NB,),
        in_specs=[
            pl.BlockSpec((_NC, _RB, _H), lambda i: (0, i, 0)),
            pl.BlockSpec((_RB, 1), lambda i: (i, 0)),
            pl.BlockSpec((_RB, _H), lambda i: (i, 0)),
            full((_H, _H)), full((_H, _H)), full((1, _H)),
        ],
        out_specs=pl.BlockSpec((_RB, _H), lambda i: (i, 0)),
        out_shape=jax.ShapeDtypeStruct((_N, _H), jnp.float32),
    )(msgp, deg, h, p['wl'], p['wr'], p['bl'].reshape(1, -1))


def _pool_body(h_ref, batch_ref, out_ref):
    i = pl.program_id(0)

    @pl.when(i == 0)
    def _():
        out_ref[...] = jnp.zeros_like(out_ref)

    b = batch_ref[0, 0, :]
    mask = (b[:, None] == lax.broadcasted_iota(jnp.int32, (1, _G), 1)
            ).astype(jnp.float32)
    out_ref[...] += lax.dot_general(
        mask, h_ref[...], (((0,), (0,)), ((), ())),
        precision=lax.Precision.HIGHEST,
        preferred_element_type=jnp.float32)


def _pool(h, batch3):
    return pl.pallas_call(
        _pool_body,
        grid=(_NB,),
        in_specs=[
            pl.BlockSpec((_RB, _H), lambda i: (i, 0)),
            pl.BlockSpec((1, 1, _RB), lambda i: (i, 0, 0)),
        ],
        out_specs=pl.BlockSpec((_G, _H), lambda i: (0, 0)),
        out_shape=jax.ShapeDtypeStruct((_G, _H), jnp.float32),
    )(h, batch3)


def _dec_mm_body(xa_ref, wa_ref, ba_ref, xb_ref, wb_ref, bb_ref,
                 oa_ref, ob_ref):
    oa_ref[...] = jnp.maximum(_dot(xa_ref[...], wa_ref[...]) + ba_ref[...],
                              0.0)
    ob_ref[...] = jnp.maximum(_dot(xb_ref[...], wb_ref[...]) + bb_ref[...],
                              0.0)


def _dec_mm(xa, pa, xb, pb):
    oa = jax.ShapeDtypeStruct((_G, pa['w'].shape[1]), jnp.float32)
    ob = jax.ShapeDtypeStruct((_G, pb['w'].shape[1]), jnp.float32)
    return pl.pallas_call(_dec_mm_body, out_shape=[oa, ob])(
        xa, pa['w'], pa['b'].reshape(1, -1), xb, pb['w'],
        pb['b'].reshape(1, -1))


def _decode(pooled, dec):
    ha, hb = dec
    xa = _ln(pooled, ha[0]['ln_g'], ha[0]['ln_b'])
    xb = _ln(pooled, hb[0]['ln_g'], hb[0]['ln_b'])
    xa, xb = _dec_mm(xa, ha[0], xb, hb[0])
    xa = _ln(xa, ha[1]['ln_g'], ha[1]['ln_b'])
    xb = _ln(xb, hb[1]['ln_g'], hb[1]['ln_b'])
    xa, xb = _dec_mm(xa, ha[1], xb, hb[1])
    return jnp.concatenate([xa, xb], axis=1)


def kernel(x, params, edge_index, batch):
    src = edge_index[0]
    dst = edge_index[1]
    r = _sc_deg(dst)
    degp = r[0] if isinstance(r, (list, tuple)) else r
    h = _encode(x, params['enc'])
    r = _sc_msg(h, src, dst)
    msgp = r[0] if isinstance(r, (list, tuple)) else r
    h, deg = _combine1(msgp, degp, h, params['convs'][0])
    for cp in params['convs'][1:]:
        r = _sc_msg(h, src, dst)
        msgp = r[0] if isinstance(r, (list, tuple)) else r
        h = _combine(msgp, deg, h, cp)
    pooled = _pool(h, batch.reshape(_NB, 1, _RB))
    return _decode(pooled, params['dec'])
